# baseline (speedup 1.0000x reference)
import functools

import jax
import jax.numpy as jnp
from jax import lax
from jax.experimental import pallas as pl
from jax.experimental.pallas import tpu as pltpu
from jax.experimental.pallas import tpu_sc as plsc

N = 10000
NP = 10240
K = 64
CH = 64
NCH = NP // CH
NCHP = 256
RT = 256
BIG = 1e30

NC, NS = 2, 16
NW = NC * NS
RW = NP // NW
RB = 32
L = 16


def _leaky(x):
    return jnp.where(x >= 0, x, 0.01 * x)


def _enc_body(x_ref, w1_ref, b1_ref, w2_ref, b2_ref, o_ref):
    h = _leaky(jnp.dot(x_ref[...], w1_ref[...],
                       preferred_element_type=jnp.float32) + b1_ref[...])
    o_ref[...] = jnp.dot(h, w2_ref[...],
                         preferred_element_type=jnp.float32) + b2_ref[...]


def _encode(xp, w1p, b1, w2, b2, rows, tile):
    return pl.pallas_call(
        _enc_body,
        grid=(rows // tile,),
        in_specs=[
            pl.BlockSpec((tile, 128), lambda i: (i, 0)),
            pl.BlockSpec((128, 32), lambda i: (0, 0)),
            pl.BlockSpec((1, 32), lambda i: (0, 0)),
            pl.BlockSpec((32, 32), lambda i: (0, 0)),
            pl.BlockSpec((1, 32), lambda i: (0, 0)),
        ],
        out_specs=pl.BlockSpec((tile, 32), lambda i: (i, 0)),
        out_shape=jax.ShapeDtypeStruct((rows, 32), jnp.float32),
    )(xp, w1p, b1.reshape(1, 32), w2, b2.reshape(1, 32))


def _dist_body(tgt_ref, src_ref, wt_ref, d_ref, cm_ref, tau_ref, a_ref):
    t = tgt_ref[...]
    s = src_ref[...]
    a_ref[...] = jnp.dot(t, wt_ref[...],
                         preferred_element_type=jnp.float32)
    tn = jnp.sum(t * t, axis=1, keepdims=True)
    sn = jnp.sum(s * s, axis=1, keepdims=True).T
    d = tn + sn - 2.0 * jnp.dot(t, s.T, preferred_element_type=jnp.float32)
    col = lax.broadcasted_iota(jnp.int32, (1, NP), 1)
    d = jnp.where(col >= N, BIG, d)
    d_ref[...] = d
    cm = jnp.min(d.reshape(RT, NCH, CH), axis=2)
    cmp_ = jnp.concatenate(
        [cm, jnp.full((RT, NCHP - NCH), BIG, jnp.float32)], axis=1)
    cm_ref[...] = cmp_
    ci = lax.broadcasted_iota(jnp.int32, (RT, NCHP), 1)

    def body(_, carry):
        x, _ = carry
        m = jnp.min(x, axis=1, keepdims=True)
        pos = jnp.min(jnp.where(x == m, ci, NCHP), axis=1, keepdims=True)
        x = jnp.where(ci == pos, BIG, x)
        return x, m

    _, tau = lax.fori_loop(0, K, body, (cmp_, jnp.zeros((RT, 1), jnp.float32)))
    tau_ref[...] = jnp.broadcast_to(tau, (RT, 128))


def _dist_tau(tgt, src, wt, hf, f):
    return pl.pallas_call(
        _dist_body,
        grid=(NP // RT,),
        in_specs=[
            pl.BlockSpec((RT, hf), lambda i: (i, 0)),
            pl.BlockSpec((NP, hf), lambda i: (0, 0)),
            pl.BlockSpec((hf, f), lambda i: (0, 0)),
        ],
        out_specs=[
            pl.BlockSpec((RT, NP), lambda i: (i, 0)),
            pl.BlockSpec((RT, NCHP), lambda i: (i, 0)),
            pl.BlockSpec((RT, 128), lambda i: (i, 0)),
            pl.BlockSpec((RT, f), lambda i: (i, 0)),
        ],
        out_shape=[
            jax.ShapeDtypeStruct((NP, NP), jnp.float32),
            jax.ShapeDtypeStruct((NP, NCHP), jnp.float32),
            jax.ShapeDtypeStruct((NP, 128), jnp.float32),
            jax.ShapeDtypeStruct((NP, f), jnp.float32),
        ],
    )(tgt, src, wt)


def _head_body(x_ref, w1_ref, b1_ref, w2_ref, b2_ref, w3_ref, b3_ref,
               w4_ref, b4_ref, o_ref):
    h = _leaky(jnp.dot(x_ref[...], w1_ref[...],
                       preferred_element_type=jnp.float32) + b1_ref[...])
    h = _leaky(jnp.dot(h, w2_ref[...],
                       preferred_element_type=jnp.float32) + b2_ref[...])
    h = _leaky(jnp.dot(h, w3_ref[...],
                       preferred_element_type=jnp.float32) + b3_ref[...])
    o_ref[...] = jnp.dot(h, w4_ref[...],
                         preferred_element_type=jnp.float32) + b4_ref[...]


def _head(x, w1, b1, w2, b2, w3, b3, w4, b4):
    def padw(w, fi, fo):
        return jnp.zeros((128, 128), jnp.float32).at[:fi, :fo].set(w)

    def padb(b):
        return jnp.zeros((1, 128), jnp.float32).at[0, :b.shape[0]].set(b)

    xp = jnp.zeros((NP, 128), jnp.float32).at[:, :x.shape[1]].set(x)
    tile = 512
    return pl.pallas_call(
        _head_body,
        grid=(NP // tile,),
        in_specs=[pl.BlockSpec((tile, 128), lambda i: (i, 0))] +
                 [pl.BlockSpec((128, 128), lambda i: (0, 0)),
                  pl.BlockSpec((1, 128), lambda i: (0, 0))] * 4,
        out_specs=pl.BlockSpec((tile, 128), lambda i: (i, 0)),
        out_shape=jax.ShapeDtypeStruct((NP, 128), jnp.float32),
    )(xp, padw(w1, 32, 64), padb(b1), padw(w2, 64, 32), padb(b2),
      padw(w3, 32, 4), padb(b3), padw(w4, 4, 1), padb(b4))


def _iota():
    return lax.iota(jnp.int32, L)


def _count(m):
    return plsc.all_reduce_population_count(m)[0]


def _perm(m):
    keys = jnp.where(m, _iota(), L + _iota())
    _, lanes = lax.sort([keys, _iota()], num_keys=1)
    return lanes


def _compact(v, lanes):
    return v.at[lanes].get(mode="promise_in_bounds")


def _bf16_rne(x):
    b = plsc.bitcast(x, jnp.int32)
    r = b + jnp.int32(0x7FFF) + ((b >> 16) & 1)
    return plsc.bitcast(r & jnp.int32(-65536), jnp.float32)


def _bcast(v, k):
    return _compact(v, jnp.full((L,), k, jnp.int32))


def _make_select_agg(HF, F):
    mesh = plsc.VectorSubcoreMesh(core_axis_name="c", subcore_axis_name="s",
                                  num_cores=NC, num_subcores=NS)
    FB = F // L
    HB = HF // L

    @functools.partial(
        pl.kernel,
        out_type=jax.ShapeDtypeStruct((NP * F,), jnp.float32),
        mesh=mesh,
        compiler_params=pltpu.CompilerParams(needs_layout_passes=False),
        scratch_types=[
            pltpu.VMEM((2, NP), jnp.float32),
            pltpu.VMEM((RB, NCHP), jnp.float32),
            pltpu.VMEM((NCHP + L,), jnp.int32),
            pltpu.VMEM((NP,), jnp.float32),
            pltpu.VMEM((NP,), jnp.int32),
            pltpu.VMEM((K,), jnp.int32),
            pltpu.VMEM((K, 128), jnp.float32),
            pltpu.VMEM((RW + L,), jnp.float32),
            pltpu.VMEM((RW * F,), jnp.float32),
            pltpu.VMEM((RW * HF,), jnp.float32),
            pltpu.VMEM((HF * F,), jnp.float32),
            pltpu.VMEM((F,), jnp.float32),
            pltpu.VMEM((RW * F,), jnp.float32),
            pltpu.SemaphoreType.DMA,
            pltpu.SemaphoreType.DMA,
        ],
    )
    def sel_agg(d_hbm, cm_hbm, tau_hbm, a_hbm, xsrc_hbm, xtgt_hbm, wb_hbm,
                bias_hbm, out_hbm,
                d_v, cm_v, chk_v, cd_v, cix_v, ix_v, g_v, tau_v, a_v, xi_v,
                w_v, bias_v, o_v, sem, dsem):
        wid = lax.axis_index("s") * NC + lax.axis_index("c")
        base = wid * RW
        pltpu.sync_copy(tau_hbm.at[pl.ds(base, RW)], tau_v.at[pl.ds(0, RW)])
        pltpu.make_async_copy(d_hbm.at[pl.ds(base * NP, NP)], d_v.at[0], dsem).start()
        pltpu.sync_copy(a_hbm.at[pl.ds(base * F, RW * F)], a_v)
        pltpu.sync_copy(xtgt_hbm.at[pl.ds(base * HF, RW * HF)], xi_v)
        pltpu.sync_copy(wb_hbm, w_v)
        pltpu.sync_copy(bias_hbm, bias_v)

        def row_block(blk, _):
            pltpu.sync_copy(cm_hbm.at[pl.ds(base + blk * RB, RB), :], cm_v)

            def row(rr, _):
                r = blk * RB + rr
                g = base + r
                par = r % 2
                pltpu.make_async_copy(d_hbm.at[pl.ds(g * NP, NP)],
                                      d_v.at[par], dsem).wait()

                @pl.when(r + 1 < RW)
                def _():
                    pltpu.make_async_copy(d_hbm.at[pl.ds((g + 1) * NP, NP)],
                                          d_v.at[1 - par], dsem).start()
                tauv = jnp.full((L,), tau_v[pl.ds(r, L)][0], jnp.float32)

                nc = jnp.int32(0)
                for gi in range(NCHP // L):
                    v = cm_v[rr, pl.ds(gi * L, L)]
                    m = v <= tauv
                    lanes = _perm(m)
                    chk_v[pl.ds(nc, L)] = lanes + gi * L
                    nc = nc + _count(m)

                def chunk(ci, mc):
                    cb = chk_v[pl.ds(ci, L)][0] * CH
                    for sub in range(CH // L):
                        v = d_v[par, pl.ds(cb + sub * L, L)]
                        m = v <= tauv
                        c16 = _count(m)

                        @pl.when(c16 > 0)
                        def _():
                            lanes = _perm(m)
                            cd_v[pl.ds(mc, L)] = _compact(v, lanes)
                            cix_v[pl.ds(mc, L)] = lanes + (cb + sub * L)
                        mc = mc + c16
                    return mc

                mc = lax.fori_loop(0, nc, chunk, jnp.int32(0))

                padn = (-mc) % L
                pm = _iota() < padn
                plsc.store_scatter(cd_v, [mc + _iota()],
                                   jnp.full((L,), BIG, jnp.float32), mask=pm)
                plsc.store_scatter(cix_v, [mc + _iota()],
                                   jnp.zeros((L,), jnp.int32), mask=pm)
                P = (mc + padn) // L

                def sort_blk(p, _):
                    sd, si = lax.sort([cd_v[pl.ds(p * L, L)],
                                       cix_v[pl.ds(p * L, L)]], num_keys=1)
                    cd_v[pl.ds(p * L, L)] = sd
                    cix_v[pl.ds(p * L, L)] = si
                    return 0

                lax.fori_loop(0, P, sort_blk, 0)

                def rnd(t, _):
                    start = t % 2

                    def pair(q, _):
                        p0 = start + 2 * q

                        @pl.when(p0 + 1 < P)
                        def _():
                            va = cd_v[pl.ds(p0 * L, L)]
                            ia = cix_v[pl.ds(p0 * L, L)]
                            vb = lax.rev(cd_v[pl.ds((p0 + 1) * L, L)], (0,))
                            ib = lax.rev(cix_v[pl.ds((p0 + 1) * L, L)], (0,))
                            lt = (va < vb) | ((va == vb) & (ia < ib))
                            lo = jnp.where(lt, va, vb)
                            loi = jnp.where(lt, ia, ib)
                            hi = jnp.where(lt, vb, va)
                            hii = jnp.where(lt, ib, ia)
                            slo, sloi = lax.sort([lo, loi], num_keys=1)
                            shi, shii = lax.sort([hi, hii], num_keys=1)
                            cd_v[pl.ds(p0 * L, L)] = slo
                            cix_v[pl.ds(p0 * L, L)] = sloi
                            cd_v[pl.ds((p0 + 1) * L, L)] = shi
                            cix_v[pl.ds((p0 + 1) * L, L)] = shii
                        return 0

                    lax.fori_loop(0, (P + 1) // 2, pair, 0)
                    return 0

                lax.fori_loop(0, P, rnd, 0)

                for kb in range(K // L):
                    ix_v[pl.ds(kb * L, L)] = cix_v[pl.ds(kb * L, L)]
                pltpu.async_copy(xsrc_hbm.at[ix_v], g_v, sem).wait()

                xi = [xi_v[pl.ds(r * HF + hb * L, L)] for hb in range(HB)]
                arow = [a_v[pl.ds(r * F + fb * L, L)] for fb in range(FB)]
                brow = [bias_v[pl.ds(fb * L, L)] for fb in range(FB)]

                def edge(j, acc):
                    s0 = list(arow)
                    s1 = [jnp.zeros((L,), jnp.float32) for _ in range(FB)]
                    s2 = [jnp.zeros((L,), jnp.float32) for _ in range(FB)]
                    s3 = [jnp.zeros((L,), jnp.float32) for _ in range(FB)]
                    ss = [s0, s1, s2, s3]
                    for hb in range(HB):
                        db = _bf16_rne(g_v[j, pl.ds(hb * L, L)] - xi[hb])
                        for k in range(L):
                            dk = _bcast(db, k)
                            s = ss[k % 4]
                            for fb in range(FB):
                                s[fb] = s[fb] + dk * w_v[
                                    pl.ds((hb * L + k) * F + fb * L, L)]
                    return tuple(
                        acc[fb] + _leaky(((s0[fb] + s1[fb]) + (s2[fb] + s3[fb]))
                                         + brow[fb])
                        for fb in range(FB))

                acc = lax.fori_loop(
                    0, K, edge,
                    tuple(jnp.zeros((L,), jnp.float32) for _ in range(FB)))
                for fb in range(FB):
                    o_v[pl.ds(r * F + fb * L, L)] = acc[fb] * (1.0 / K)
                return 0

            lax.fori_loop(0, RB, row, 0)
            return 0

        lax.fori_loop(0, RW // RB, row_block, 0)
        pltpu.sync_copy(o_v, out_hbm.at[pl.ds(base * F, RW * F)])

    return sel_agg


def _make_gather(Fout):
    mesh = plsc.VectorSubcoreMesh(core_axis_name="c", subcore_axis_name="s",
                                  num_cores=NC, num_subcores=NS)

    @functools.partial(
        pl.kernel,
        out_type=jax.ShapeDtypeStruct((NP, 128), jnp.float32),
        mesh=mesh,
        compiler_params=pltpu.CompilerParams(needs_layout_passes=False),
        scratch_types=[
            pltpu.VMEM((RW,), jnp.int32),
            pltpu.VMEM((RW, 128), jnp.float32),
            pltpu.SemaphoreType.DMA,
        ],
    )
    def gat(table_hbm, idx_hbm, out_hbm, idx_v, rows_v, sem):
        wid = lax.axis_index("s") * NC + lax.axis_index("c")
        base = wid * RW
        pltpu.sync_copy(idx_hbm.at[pl.ds(base, RW)], idx_v)
        pltpu.async_copy(table_hbm.at[idx_v], rows_v, sem).wait()
        pltpu.sync_copy(rows_v, out_hbm.at[pl.ds(base, RW)])

    return gat


def _pad128(x):
    return jnp.zeros((x.shape[0], 128), jnp.float32).at[:, :x.shape[1]].set(x)


def _select_aggregate(d, cm, tau, a, xsrc_pad, xtgt, wb, bias, hf, f):
    fn = _make_select_agg(hf, f)
    wbr = wb.astype(jnp.bfloat16).astype(jnp.float32)
    out = fn(d.reshape(-1), cm, tau[:, 0], a.reshape(-1),
             xsrc_pad, xtgt.reshape(-1), wbr.reshape(-1), bias)
    return out.reshape(NP, f)


def _gather_rows_pad(table_pad, idx):
    return _make_gather(128)(table_pad, idx)


def kernel(x_pfc, x_vtx, pfc_w1, pfc_b1, pfc_w2, pfc_b2, vtx_w1, vtx_b1,
           vtx_w2, vtx_b2, conv_w, conv_b, conv2_w, conv2_b, out_w1, out_b1,
           out_w2, out_b2, out_w3, out_b3, out_w4, out_b4, batch_pfc,
           batch_vtx):
    H = 32
    xp = jnp.zeros((NP, 128), jnp.float32).at[:N, :13].set(x_pfc)
    w1p = jnp.zeros((128, 32), jnp.float32).at[:13, :].set(pfc_w1)
    enc = _encode(xp, w1p, pfc_b1, pfc_w2, pfc_b2, NP, 512)

    xv = jnp.zeros((1024, 128), jnp.float32).at[:, :4].set(x_vtx)
    wv1p = jnp.zeros((128, 32), jnp.float32).at[:4, :].set(vtx_w1)
    x_vtx_enc = _encode(xv, wv1p, vtx_b1, vtx_w2, vtx_b2, 1024, 512)

    d1, cm1, tau1, a1 = _dist_tau(enc, enc, conv_w[:H], H, 16)
    feats1p = _select_aggregate(d1, cm1, tau1, a1, _pad128(enc), enc,
                                conv_w[H:], conv_b, H, 16)

    charged_idx = jnp.nonzero(x_pfc[:, 11] != 0, size=N, fill_value=0)[0]
    cidx = jnp.concatenate(
        [charged_idx.astype(jnp.int32), jnp.zeros((NP - N,), jnp.int32)])
    charged_pad = _gather_rows_pad(_pad128(feats1p), cidx)
    charged = charged_pad[:, :16]

    d2, cm2, tau2, a2 = _dist_tau(feats1p, charged, conv2_w[:16], 16, 32)
    feats2p = _select_aggregate(d2, cm2, tau2, a2, charged_pad, feats1p,
                                conv2_w[16:], conv2_b, 16, 32)

    outp = _head(feats2p, out_w1, out_b1, out_w2, out_b2, out_w3, out_b3,
                 out_w4, out_b4)

    return (outp[:N, :1], batch_pfc, feats1p[:N], x_vtx_enc)


# speedup vs baseline: 1.1695x; 1.1695x over previous
import functools

import jax
import jax.numpy as jnp
from jax import lax
from jax.experimental import pallas as pl
from jax.experimental.pallas import tpu as pltpu
from jax.experimental.pallas import tpu_sc as plsc

N = 10000
NP = 10240
K = 64
CH = 64
NCH = NP // CH
NCHP = 256
RT = 256
BIG = 1e30

NC, NS = 2, 16
NW = NC * NS
RW = NP // NW
RB = 32
L = 16


def _leaky(x):
    return jnp.where(x >= 0, x, 0.01 * x)


def _enc_body(x_ref, w1_ref, b1_ref, w2_ref, b2_ref, o_ref):
    h = _leaky(jnp.dot(x_ref[...], w1_ref[...],
                       preferred_element_type=jnp.float32) + b1_ref[...])
    o_ref[...] = jnp.dot(h, w2_ref[...],
                         preferred_element_type=jnp.float32) + b2_ref[...]


def _encode(xp, w1p, b1, w2, b2, rows, tile):
    return pl.pallas_call(
        _enc_body,
        grid=(rows // tile,),
        in_specs=[
            pl.BlockSpec((tile, 128), lambda i: (i, 0)),
            pl.BlockSpec((128, 32), lambda i: (0, 0)),
            pl.BlockSpec((1, 32), lambda i: (0, 0)),
            pl.BlockSpec((32, 32), lambda i: (0, 0)),
            pl.BlockSpec((1, 32), lambda i: (0, 0)),
        ],
        out_specs=pl.BlockSpec((tile, 32), lambda i: (i, 0)),
        out_shape=jax.ShapeDtypeStruct((rows, 32), jnp.float32),
    )(xp, w1p, b1.reshape(1, 32), w2, b2.reshape(1, 32))


def _dist_body(tgt_ref, src_ref, wt_ref, d_ref, cm_ref, tau_ref, a_ref):
    t = tgt_ref[...]
    s = src_ref[...]
    a_ref[...] = jnp.dot(t, wt_ref[...],
                         preferred_element_type=jnp.float32)
    tn = jnp.sum(t * t, axis=1, keepdims=True)
    sn = jnp.sum(s * s, axis=1, keepdims=True).T
    d = tn + sn - 2.0 * jnp.dot(t, s.T, preferred_element_type=jnp.float32)
    col = lax.broadcasted_iota(jnp.int32, (1, NP), 1)
    d = jnp.where(col >= N, BIG, d)
    d_ref[...] = d
    cm = jnp.min(d.reshape(RT, NCH, CH), axis=2)
    cmp_ = jnp.concatenate(
        [cm, jnp.full((RT, NCHP - NCH), BIG, jnp.float32)], axis=1)
    cm_ref[...] = cmp_
    ci = lax.broadcasted_iota(jnp.int32, (RT, NCHP), 1)

    def body(_, carry):
        x, _ = carry
        m = jnp.min(x, axis=1, keepdims=True)
        pos = jnp.min(jnp.where(x == m, ci, NCHP), axis=1, keepdims=True)
        x = jnp.where(ci == pos, BIG, x)
        return x, m

    _, tau = lax.fori_loop(0, K, body, (cmp_, jnp.zeros((RT, 1), jnp.float32)))
    tau_ref[...] = jnp.broadcast_to(tau, (RT, 128))


def _dist_tau(tgt, src, wt, hf, f):
    return pl.pallas_call(
        _dist_body,
        grid=(NP // RT,),
        in_specs=[
            pl.BlockSpec((RT, hf), lambda i: (i, 0)),
            pl.BlockSpec((NP, hf), lambda i: (0, 0)),
            pl.BlockSpec((hf, f), lambda i: (0, 0)),
        ],
        out_specs=[
            pl.BlockSpec((RT, NP), lambda i: (i, 0)),
            pl.BlockSpec((RT, NCHP), lambda i: (i, 0)),
            pl.BlockSpec((RT, 128), lambda i: (i, 0)),
            pl.BlockSpec((RT, f), lambda i: (i, 0)),
        ],
        out_shape=[
            jax.ShapeDtypeStruct((NP, NP), jnp.float32),
            jax.ShapeDtypeStruct((NP, NCHP), jnp.float32),
            jax.ShapeDtypeStruct((NP, 128), jnp.float32),
            jax.ShapeDtypeStruct((NP, f), jnp.float32),
        ],
    )(tgt, src, wt)


def _head_body(x_ref, w1_ref, b1_ref, w2_ref, b2_ref, w3_ref, b3_ref,
               w4_ref, b4_ref, o_ref):
    h = _leaky(jnp.dot(x_ref[...], w1_ref[...],
                       preferred_element_type=jnp.float32) + b1_ref[...])
    h = _leaky(jnp.dot(h, w2_ref[...],
                       preferred_element_type=jnp.float32) + b2_ref[...])
    h = _leaky(jnp.dot(h, w3_ref[...],
                       preferred_element_type=jnp.float32) + b3_ref[...])
    o_ref[...] = jnp.dot(h, w4_ref[...],
                         preferred_element_type=jnp.float32) + b4_ref[...]


def _head(x, w1, b1, w2, b2, w3, b3, w4, b4):
    def padw(w, fi, fo):
        return jnp.zeros((128, 128), jnp.float32).at[:fi, :fo].set(w)

    def padb(b):
        return jnp.zeros((1, 128), jnp.float32).at[0, :b.shape[0]].set(b)

    xp = jnp.zeros((NP, 128), jnp.float32).at[:, :x.shape[1]].set(x)
    tile = 512
    return pl.pallas_call(
        _head_body,
        grid=(NP // tile,),
        in_specs=[pl.BlockSpec((tile, 128), lambda i: (i, 0))] +
                 [pl.BlockSpec((128, 128), lambda i: (0, 0)),
                  pl.BlockSpec((1, 128), lambda i: (0, 0))] * 4,
        out_specs=pl.BlockSpec((tile, 128), lambda i: (i, 0)),
        out_shape=jax.ShapeDtypeStruct((NP, 128), jnp.float32),
    )(xp, padw(w1, 32, 64), padb(b1), padw(w2, 64, 32), padb(b2),
      padw(w3, 32, 4), padb(b3), padw(w4, 4, 1), padb(b4))


def _iota():
    return lax.iota(jnp.int32, L)


def _count(m):
    return plsc.all_reduce_population_count(m)[0]


def _perm(m):
    keys = jnp.where(m, _iota(), L + _iota())
    _, lanes = lax.sort([keys, _iota()], num_keys=1)
    return lanes


def _compact(v, lanes):
    return v.at[lanes].get(mode="promise_in_bounds")


def _bf16_rne(x):
    b = plsc.bitcast(x, jnp.int32)
    r = b + jnp.int32(0x7FFF) + ((b >> 16) & 1)
    return plsc.bitcast(r & jnp.int32(-65536), jnp.float32)


def _bcast(v, k):
    return _compact(v, jnp.full((L,), k, jnp.int32))


def _make_select_agg(HF, F):
    mesh = plsc.VectorSubcoreMesh(core_axis_name="c", subcore_axis_name="s",
                                  num_cores=NC, num_subcores=NS)
    FB = F // L
    HB = HF // L

    @functools.partial(
        pl.kernel,
        out_type=jax.ShapeDtypeStruct((NP * F,), jnp.float32),
        mesh=mesh,
        compiler_params=pltpu.CompilerParams(needs_layout_passes=False),
        scratch_types=[
            pltpu.VMEM((NP,), jnp.float32),
            pltpu.VMEM((RB * NCHP,), jnp.float32),
            pltpu.VMEM((NCHP + L,), jnp.int32),
            pltpu.VMEM((NP,), jnp.float32),
            pltpu.VMEM((NP,), jnp.int32),
            pltpu.VMEM((K,), jnp.int32),
            pltpu.VMEM((K, 128), jnp.float32),
            pltpu.VMEM((RW + L,), jnp.float32),
            pltpu.VMEM((RW * F,), jnp.float32),
            pltpu.VMEM((RW * HF,), jnp.float32),
            pltpu.VMEM((HF * F,), jnp.float32),
            pltpu.VMEM((F,), jnp.float32),
            pltpu.VMEM((RW * F,), jnp.float32),
            pltpu.SemaphoreType.DMA,
        ],
    )
    def sel_agg(d_hbm, cm_hbm, tau_hbm, a_hbm, xsrc_hbm, xtgt_hbm, wb_hbm,
                bias_hbm, out_hbm,
                d_v, cm_v, chk_v, cd_v, cix_v, ix_v, g_v, tau_v, a_v, xi_v,
                w_v, bias_v, o_v, sem):
        wid = lax.axis_index("s") * NC + lax.axis_index("c")
        base = wid * RW
        pltpu.sync_copy(tau_hbm.at[pl.ds(base, RW)], tau_v.at[pl.ds(0, RW)])
        pltpu.sync_copy(a_hbm.at[pl.ds(base * F, RW * F)], a_v)
        pltpu.sync_copy(xtgt_hbm.at[pl.ds(base * HF, RW * HF)], xi_v)
        pltpu.sync_copy(wb_hbm, w_v)
        pltpu.sync_copy(bias_hbm, bias_v)

        def row_block(blk, _):
            pltpu.sync_copy(
                cm_hbm.at[pl.ds((base + blk * RB) * NCHP, RB * NCHP)], cm_v)

            def row(rr, _):
                r = blk * RB + rr
                g = base + r
                pltpu.sync_copy(d_hbm.at[pl.ds(g * NP, NP)], d_v)
                tauv = jnp.full((L,), tau_v[pl.ds(r, L)][0], jnp.float32)

                nc = jnp.int32(0)
                for gi in range(NCHP // L):
                    v = cm_v[pl.ds(rr * NCHP + gi * L, L)]
                    m = v <= tauv
                    lanes = _perm(m)
                    chk_v[pl.ds(nc, L)] = lanes + gi * L
                    nc = nc + _count(m)

                def chunk(ci, mc):
                    cb = chk_v[pl.ds(ci, L)][0] * CH
                    for sub in range(CH // L):
                        v = d_v[pl.ds(cb + sub * L, L)]
                        m = v <= tauv
                        lanes = _perm(m)
                        cd_v[pl.ds(mc, L)] = _compact(v, lanes)
                        cix_v[pl.ds(mc, L)] = lanes + (cb + sub * L)
                        mc = mc + _count(m)
                    return mc

                mc = lax.fori_loop(0, nc, chunk, jnp.int32(0))

                padn = (-mc) % L
                pm = _iota() < padn
                plsc.store_scatter(cd_v, [mc + _iota()],
                                   jnp.full((L,), BIG, jnp.float32), mask=pm)
                plsc.store_scatter(cix_v, [mc + _iota()],
                                   jnp.zeros((L,), jnp.int32), mask=pm)
                P = (mc + padn) // L

                def sort_blk(p, _):
                    sd, si = lax.sort([cd_v[pl.ds(p * L, L)],
                                       cix_v[pl.ds(p * L, L)]], num_keys=1)
                    cd_v[pl.ds(p * L, L)] = sd
                    cix_v[pl.ds(p * L, L)] = si
                    return 0

                lax.fori_loop(0, P, sort_blk, 0)

                def rnd(t, _):
                    start = t % 2

                    def pair(q, _):
                        p0 = start + 2 * q

                        @pl.when(p0 + 1 < P)
                        def _():
                            va = cd_v[pl.ds(p0 * L, L)]
                            ia = cix_v[pl.ds(p0 * L, L)]
                            vb = lax.rev(cd_v[pl.ds((p0 + 1) * L, L)], (0,))
                            ib = lax.rev(cix_v[pl.ds((p0 + 1) * L, L)], (0,))
                            lt = (va < vb) | ((va == vb) & (ia < ib))
                            lo = jnp.where(lt, va, vb)
                            loi = jnp.where(lt, ia, ib)
                            hi = jnp.where(lt, vb, va)
                            hii = jnp.where(lt, ib, ia)
                            slo, sloi = lax.sort([lo, loi], num_keys=1)
                            shi, shii = lax.sort([hi, hii], num_keys=1)
                            cd_v[pl.ds(p0 * L, L)] = slo
                            cix_v[pl.ds(p0 * L, L)] = sloi
                            cd_v[pl.ds((p0 + 1) * L, L)] = shi
                            cix_v[pl.ds((p0 + 1) * L, L)] = shii
                        return 0

                    lax.fori_loop(0, (P + 1) // 2, pair, 0)
                    return 0

                lax.fori_loop(0, P, rnd, 0)

                for kb in range(K // L):
                    ix_v[pl.ds(kb * L, L)] = cix_v[pl.ds(kb * L, L)]
                pltpu.async_copy(xsrc_hbm.at[ix_v], g_v, sem).wait()

                xi = [xi_v[pl.ds(r * HF + hb * L, L)] for hb in range(HB)]
                arow = [a_v[pl.ds(r * F + fb * L, L)] for fb in range(FB)]
                brow = [bias_v[pl.ds(fb * L, L)] for fb in range(FB)]

                def edge(j, acc):
                    s = list(arow)
                    for hb in range(HB):
                        db = _bf16_rne(g_v[j, pl.ds(hb * L, L)] - xi[hb])
                        for k in range(L):
                            dk = _bcast(db, k)
                            for fb in range(FB):
                                s[fb] = s[fb] + dk * w_v[
                                    pl.ds((hb * L + k) * F + fb * L, L)]
                    return tuple(acc[fb] + _leaky(s[fb] + brow[fb])
                                 for fb in range(FB))

                acc = lax.fori_loop(
                    0, K, edge,
                    tuple(jnp.zeros((L,), jnp.float32) for _ in range(FB)))
                for fb in range(FB):
                    o_v[pl.ds(r * F + fb * L, L)] = acc[fb] * (1.0 / K)
                return 0

            lax.fori_loop(0, RB, row, 0)
            return 0

        lax.fori_loop(0, RW // RB, row_block, 0)
        pltpu.sync_copy(o_v, out_hbm.at[pl.ds(base * F, RW * F)])

    return sel_agg


def _make_gather(Fout):
    mesh = plsc.VectorSubcoreMesh(core_axis_name="c", subcore_axis_name="s",
                                  num_cores=NC, num_subcores=NS)

    @functools.partial(
        pl.kernel,
        out_type=jax.ShapeDtypeStruct((NP, 128), jnp.float32),
        mesh=mesh,
        compiler_params=pltpu.CompilerParams(needs_layout_passes=False),
        scratch_types=[
            pltpu.VMEM((RW,), jnp.int32),
            pltpu.VMEM((RW, 128), jnp.float32),
            pltpu.SemaphoreType.DMA,
        ],
    )
    def gat(table_hbm, idx_hbm, out_hbm, idx_v, rows_v, sem):
        wid = lax.axis_index("s") * NC + lax.axis_index("c")
        base = wid * RW
        pltpu.sync_copy(idx_hbm.at[pl.ds(base, RW)], idx_v)
        pltpu.async_copy(table_hbm.at[idx_v], rows_v, sem).wait()
        pltpu.sync_copy(rows_v, out_hbm.at[pl.ds(base, RW)])

    return gat


def _pad128(x):
    return jnp.zeros((x.shape[0], 128), jnp.float32).at[:, :x.shape[1]].set(x)


def _select_aggregate(d, cm, tau, a, xsrc_pad, xtgt, wb, bias, hf, f):
    fn = _make_select_agg(hf, f)
    wbr = wb.astype(jnp.bfloat16).astype(jnp.float32)
    out = fn(d.reshape(-1), cm.reshape(-1), tau[:, 0], a.reshape(-1),
             xsrc_pad, xtgt.reshape(-1), wbr.reshape(-1), bias)
    return out.reshape(NP, f)


def _gather_rows_pad(table_pad, idx):
    return _make_gather(128)(table_pad, idx)


def kernel(x_pfc, x_vtx, pfc_w1, pfc_b1, pfc_w2, pfc_b2, vtx_w1, vtx_b1,
           vtx_w2, vtx_b2, conv_w, conv_b, conv2_w, conv2_b, out_w1, out_b1,
           out_w2, out_b2, out_w3, out_b3, out_w4, out_b4, batch_pfc,
           batch_vtx):
    H = 32
    xp = jnp.zeros((NP, 128), jnp.float32).at[:N, :13].set(x_pfc)
    w1p = jnp.zeros((128, 32), jnp.float32).at[:13, :].set(pfc_w1)
    enc = _encode(xp, w1p, pfc_b1, pfc_w2, pfc_b2, NP, 512)

    xv = jnp.zeros((1024, 128), jnp.float32).at[:, :4].set(x_vtx)
    wv1p = jnp.zeros((128, 32), jnp.float32).at[:4, :].set(vtx_w1)
    x_vtx_enc = _encode(xv, wv1p, vtx_b1, vtx_w2, vtx_b2, 1024, 512)

    d1, cm1, tau1, a1 = _dist_tau(enc, enc, conv_w[:H], H, 16)
    feats1p = _select_aggregate(d1, cm1, tau1, a1, _pad128(enc), enc,
                                conv_w[H:], conv_b, H, 16)

    charged_idx = jnp.nonzero(x_pfc[:, 11] != 0, size=N, fill_value=0)[0]
    cidx = jnp.concatenate(
        [charged_idx.astype(jnp.int32), jnp.zeros((NP - N,), jnp.int32)])
    charged_pad = _gather_rows_pad(_pad128(feats1p), cidx)
    charged = charged_pad[:, :16]

    d2, cm2, tau2, a2 = _dist_tau(feats1p, charged, conv2_w[:16], 16, 32)
    feats2p = _select_aggregate(d2, cm2, tau2, a2, charged_pad, feats1p,
                                conv2_w[16:], conv2_b, 16, 32)

    outp = _head(feats2p, out_w1, out_b1, out_w2, out_b2, out_w3, out_b3,
                 out_w4, out_b4)

    return (outp[:N, :1], batch_pfc, feats1p[:N], x_vtx_enc)
